# baseline (speedup 1.0000x reference)
import functools

import jax
import jax.numpy as jnp
from jax import lax
from jax.experimental import pallas as pl
from jax.experimental.pallas import tpu as pltpu
from jax.experimental.pallas import tpu_sc as plsc

N = 10000
E = 320000
D = 128
WEXT = 144

NCORES = 2
NSUB = 16
NW = NCORES * NSUB
CHUNK = 64
CH = 160
EPW = CH * CHUNK
N_ACC = 10112
ROWS_PER_TILE = N_ACC // NSUB


def _make_sc_agg(width, nbuf):
  mesh = plsc.VectorSubcoreMesh(core_axis_name="c", subcore_axis_name="s")

  @functools.partial(
      pl.kernel,
      out_type=jax.ShapeDtypeStruct((NCORES, N_ACC, width), jnp.float32),
      mesh=mesh,
      compiler_params=pltpu.CompilerParams(use_tc_tiling_on_sc=False),
      scratch_types=[
          pltpu.VMEM((CH, CHUNK), jnp.int32),
          pltpu.VMEM((CH, CHUNK), jnp.int32),
          pltpu.VMEM((nbuf, CHUNK, width), jnp.float32),
          pltpu.VMEM_SHARED((N_ACC, width), jnp.float32),
      ] + [pltpu.SemaphoreType.DMA] * (2 * nbuf),
  )
  def sc_agg(h_hbm, src_hbm, dst_hbm, z_hbm, out_hbm, sidx, didx, msgs,
             acc, *sems):
    gsem = sems[:nbuf]
    ssem = sems[nbuf:]
    c = lax.axis_index("c")
    s = lax.axis_index("s")
    wid = s * NCORES + c

    pltpu.sync_copy(z_hbm, acc.at[pl.ds(s * ROWS_PER_TILE, ROWS_PER_TILE)])
    pltpu.sync_copy(src_hbm.at[wid], sidx)
    pltpu.sync_copy(dst_hbm.at[wid], didx)
    plsc.subcore_barrier()

    for b in range(nbuf):
      pltpu.async_copy(h_hbm.at[sidx.at[b]], msgs.at[b], gsem[b])

    def group_body(g, _):
      j0 = g * nbuf
      scat = []
      for b in range(nbuf):
        pltpu.make_async_copy(h_hbm.at[sidx.at[b]], msgs.at[b],
                              gsem[b]).wait()
        scat.append(pltpu.async_copy(msgs.at[b], acc.at[didx.at[j0 + b]],
                                     ssem[b], add=True))
      for b in range(nbuf):
        scat[b].wait()
        pltpu.async_copy(h_hbm.at[sidx.at[j0 + nbuf + b]], msgs.at[b],
                         gsem[b])
      return ()

    lax.fori_loop(0, CH // nbuf - 1, group_body, ())
    scat = []
    for b in range(nbuf):
      j = CH - nbuf + b
      pltpu.make_async_copy(h_hbm.at[sidx.at[b]], msgs.at[b], gsem[b]).wait()
      scat.append(pltpu.async_copy(msgs.at[b], acc.at[didx.at[j]], ssem[b],
                                   add=True))
    for b in range(nbuf):
      scat[b].wait()
    plsc.subcore_barrier()

    r0 = s * ROWS_PER_TILE
    pltpu.sync_copy(acc.at[pl.ds(r0, ROWS_PER_TILE)],
                    out_hbm.at[c, pl.ds(r0, ROWS_PER_TILE)])

  return sc_agg


_sc_agg_ext = _make_sc_agg(WEXT, 2)
_sc_agg_d = _make_sc_agg(D, 2)

_TC_R = 400


def _tc1_body(h_ref, p_ref, ws_ref, wn_ref, b_ref, g_ref, bb_ref,
              o_ref, d_ref):
  p = p_ref[...]
  ssum = p[0] + p[1]
  agg = ssum[:, :D]
  dinv = 1.0 / jnp.maximum(ssum[:, D], 1.0)
  agg = agg * dinv[:, None]
  out = (jnp.dot(h_ref[...], ws_ref[...], preferred_element_type=jnp.float32)
         + jnp.dot(agg, wn_ref[...], preferred_element_type=jnp.float32)
         + b_ref[...])
  mu = jnp.mean(out, axis=-1, keepdims=True)
  var = jnp.mean((out - mu) * (out - mu), axis=-1, keepdims=True)
  y = (out - mu) * lax.rsqrt(var + 1e-5) * g_ref[...] + bb_ref[...]
  o_ref[...] = jnp.maximum(y, 0.0)
  d_ref[...] = dinv[:, None]


def _tc2_body(h_ref, p_ref, d_ref, ws_ref, wn_ref, b_ref, o_ref):
  p = p_ref[...]
  agg = (p[0] + p[1]) * d_ref[...]
  o_ref[...] = (
      jnp.dot(h_ref[...], ws_ref[...], preferred_element_type=jnp.float32)
      + jnp.dot(agg, wn_ref[...], preferred_element_type=jnp.float32)
      + b_ref[...])


def _full(shape):
  return pl.BlockSpec(shape, lambda i: (0,) * len(shape))


def _tc_layer1(feat, parts, W_self, W_neigh, b, g, beta):
  grid = (N // _TC_R,)
  return pl.pallas_call(
      _tc1_body,
      grid=grid,
      in_specs=[
          pl.BlockSpec((_TC_R, D), lambda i: (i, 0)),
          pl.BlockSpec((NCORES, _TC_R, WEXT), lambda i: (0, i, 0)),
          _full((D, D)),
          _full((D, D)),
          _full((1, D)),
          _full((1, D)),
          _full((1, D)),
      ],
      out_specs=[
          pl.BlockSpec((_TC_R, D), lambda i: (i, 0)),
          pl.BlockSpec((_TC_R, 1), lambda i: (i, 0)),
      ],
      out_shape=[
          jax.ShapeDtypeStruct((N, D), jnp.float32),
          jax.ShapeDtypeStruct((N, 1), jnp.float32),
      ],
  )(feat, parts, W_self, W_neigh, b.reshape(1, D), g.reshape(1, D),
    beta.reshape(1, D))


def _tc_layer2(h, parts, dinv, W_self, W_neigh, b):
  grid = (N // _TC_R,)
  return pl.pallas_call(
      _tc2_body,
      grid=grid,
      in_specs=[
          pl.BlockSpec((_TC_R, D), lambda i: (i, 0)),
          pl.BlockSpec((NCORES, _TC_R, D), lambda i: (0, i, 0)),
          pl.BlockSpec((_TC_R, 1), lambda i: (i, 0)),
          _full((D, D)),
          _full((D, D)),
          _full((1, D)),
      ],
      out_specs=pl.BlockSpec((_TC_R, D), lambda i: (i, 0)),
      out_shape=jax.ShapeDtypeStruct((N, D), jnp.float32),
  )(h, parts, dinv, W_self, W_neigh, b.reshape(1, D))


def kernel(feat, edge_index, W_self0, W_neigh0, b0, W_self1, W_neigh1, b1,
           ln_g, ln_b):
  epw_real = E // NW
  pad = EPW - epw_real
  src = jnp.pad(edge_index[0].reshape(NW, epw_real), ((0, 0), (0, pad)),
                constant_values=0).reshape(NW, CH, CHUNK)
  dst = jnp.pad(edge_index[1].reshape(NW, epw_real), ((0, 0), (0, pad)),
                constant_values=N).reshape(NW, CH, CHUNK)

  feat_ext = jnp.concatenate(
      [feat, jnp.ones((N, 1), jnp.float32), jnp.zeros((N, WEXT - D - 1),
                                                      jnp.float32)], axis=1)
  z_ext = jnp.zeros((ROWS_PER_TILE, WEXT), jnp.float32)
  z_d = jnp.zeros((ROWS_PER_TILE, D), jnp.float32)

  parts1 = _sc_agg_ext(feat_ext, src, dst, z_ext)
  h1, dinv = _tc_layer1(feat, parts1, W_self0, W_neigh0, b0, ln_g, ln_b)
  parts2 = _sc_agg_d(h1, src, dst, z_d)
  return _tc_layer2(h1, parts2, dinv, W_self1, W_neigh1, b1)


# speedup vs baseline: 1.1120x; 1.1120x over previous
import functools

import jax
import jax.numpy as jnp
from jax import lax
from jax.experimental import pallas as pl
from jax.experimental.pallas import tpu as pltpu
from jax.experimental.pallas import tpu_sc as plsc

N = 10000
E = 320000
D = 128
WEXT = 144

NCORES = 2
NSUB = 16
NW = NCORES * NSUB
CHUNK = 128
CH = 80
EPW = CH * CHUNK
HALF = 2
N_ACC = 10112
ROWS_PER_TILE = N_ACC // NSUB


def _make_sc_agg(width, nbuf):
  mesh = plsc.VectorSubcoreMesh(core_axis_name="c", subcore_axis_name="s")

  @functools.partial(
      pl.kernel,
      out_type=jax.ShapeDtypeStruct((NCORES, N_ACC, width), jnp.float32),
      mesh=mesh,
      compiler_params=pltpu.CompilerParams(use_tc_tiling_on_sc=False),
      scratch_types=[
          pltpu.VMEM((HALF, CHUNK), jnp.int32),
          pltpu.VMEM((HALF, CHUNK), jnp.int32),
          pltpu.VMEM((HALF, CHUNK), jnp.int32),
          pltpu.VMEM((HALF, CHUNK), jnp.int32),
          pltpu.VMEM((2, CHUNK, width), jnp.float32),
          pltpu.VMEM_SHARED((N_ACC, width), jnp.float32),
      ] + [pltpu.SemaphoreType.DMA] * 4,
  )
  def sc_agg(h_hbm, src_hbm, dst_hbm, z_hbm, out_hbm, srcb0, srcb1, dstb0,
             dstb1, msgs, acc, gsem0, gsem1, isem0, isem1):
    gsem = (gsem0, gsem1)
    isem = (isem0, isem1)
    srcb = (srcb0, srcb1)
    dstb = (dstb0, dstb1)
    c = lax.axis_index("c")
    s = lax.axis_index("s")
    wid = s * NCORES + c

    def fire_idx(p, c0):
      pltpu.async_copy(src_hbm.at[wid, pl.ds(c0, HALF)], srcb[p], isem[p])
      pltpu.async_copy(dst_hbm.at[wid, pl.ds(c0, HALF)], dstb[p], isem[p])

    def drain_idx(p):
      pltpu.make_async_copy(src_hbm.at[wid, pl.ds(0, HALF)], srcb[p],
                            isem[p]).wait()
      pltpu.make_async_copy(dst_hbm.at[wid, pl.ds(0, HALF)], dstb[p],
                            isem[p]).wait()

    def wait_gather(b):
      pltpu.make_async_copy(h_hbm.at[srcb0.at[0]], msgs.at[b],
                            gsem[b]).wait()

    pltpu.sync_copy(z_hbm, acc.at[pl.ds(s * ROWS_PER_TILE, ROWS_PER_TILE)])
    pltpu.sync_copy(src_hbm.at[wid, pl.ds(0, HALF)], srcb0)
    pltpu.sync_copy(dst_hbm.at[wid, pl.ds(0, HALF)], dstb0)
    plsc.subcore_barrier()
    fire_idx(1, 2)
    for b in range(2):
      pltpu.async_copy(h_hbm.at[srcb0.at[b]], msgs.at[b], gsem[b])

    def sg_body(sg, _):
      j0 = 4 * sg
      drain_idx(1)
      for b in range(2):
        wait_gather(b)
        pltpu.sync_copy(msgs.at[b], acc.at[dstb0.at[b]], add=True)
        pltpu.async_copy(h_hbm.at[srcb1.at[b]], msgs.at[b], gsem[b])
      fire_idx(0, j0 + 4)
      for b in range(2):
        wait_gather(b)
        pltpu.sync_copy(msgs.at[b], acc.at[dstb1.at[b]], add=True)
        if b == 0:
          drain_idx(0)
        pltpu.async_copy(h_hbm.at[srcb0.at[b]], msgs.at[b], gsem[b])
      fire_idx(1, j0 + 6)
      return ()

    lax.fori_loop(0, CH // 4 - 1, sg_body, ())

    drain_idx(1)
    for b in range(2):
      wait_gather(b)
      pltpu.sync_copy(msgs.at[b], acc.at[dstb0.at[b]], add=True)
      pltpu.async_copy(h_hbm.at[srcb1.at[b]], msgs.at[b], gsem[b])
    for b in range(2):
      wait_gather(b)
      pltpu.sync_copy(msgs.at[b], acc.at[dstb1.at[b]], add=True)
    plsc.subcore_barrier()

    r0 = s * ROWS_PER_TILE
    pltpu.sync_copy(acc.at[pl.ds(r0, ROWS_PER_TILE)],
                    out_hbm.at[c, pl.ds(r0, ROWS_PER_TILE)])

  return sc_agg


_sc_agg_ext = _make_sc_agg(WEXT, 2)
_sc_agg_d = _make_sc_agg(D, 2)

_TC_R = 400


def _tc1_body(h_ref, p_ref, ws_ref, wn_ref, b_ref, g_ref, bb_ref,
              o_ref, d_ref):
  p = p_ref[...]
  ssum = p[0] + p[1]
  agg = ssum[:, :D]
  dinv = 1.0 / jnp.maximum(ssum[:, D], 1.0)
  agg = agg * dinv[:, None]
  out = (jnp.dot(h_ref[...], ws_ref[...], preferred_element_type=jnp.float32)
         + jnp.dot(agg, wn_ref[...], preferred_element_type=jnp.float32)
         + b_ref[...])
  mu = jnp.mean(out, axis=-1, keepdims=True)
  var = jnp.mean((out - mu) * (out - mu), axis=-1, keepdims=True)
  y = (out - mu) * lax.rsqrt(var + 1e-5) * g_ref[...] + bb_ref[...]
  o_ref[...] = jnp.maximum(y, 0.0)
  d_ref[...] = dinv[:, None]


def _tc2_body(h_ref, p_ref, d_ref, ws_ref, wn_ref, b_ref, o_ref):
  p = p_ref[...]
  agg = (p[0] + p[1]) * d_ref[...]
  o_ref[...] = (
      jnp.dot(h_ref[...], ws_ref[...], preferred_element_type=jnp.float32)
      + jnp.dot(agg, wn_ref[...], preferred_element_type=jnp.float32)
      + b_ref[...])


def _full(shape):
  return pl.BlockSpec(shape, lambda i: (0,) * len(shape))


def _tc_layer1(feat, parts, W_self, W_neigh, b, g, beta):
  grid = (N // _TC_R,)
  return pl.pallas_call(
      _tc1_body,
      grid=grid,
      in_specs=[
          pl.BlockSpec((_TC_R, D), lambda i: (i, 0)),
          pl.BlockSpec((NCORES, _TC_R, WEXT), lambda i: (0, i, 0)),
          _full((D, D)),
          _full((D, D)),
          _full((1, D)),
          _full((1, D)),
          _full((1, D)),
      ],
      out_specs=[
          pl.BlockSpec((_TC_R, D), lambda i: (i, 0)),
          pl.BlockSpec((_TC_R, 1), lambda i: (i, 0)),
      ],
      out_shape=[
          jax.ShapeDtypeStruct((N, D), jnp.float32),
          jax.ShapeDtypeStruct((N, 1), jnp.float32),
      ],
  )(feat, parts, W_self, W_neigh, b.reshape(1, D), g.reshape(1, D),
    beta.reshape(1, D))


def _tc_layer2(h, parts, dinv, W_self, W_neigh, b):
  grid = (N // _TC_R,)
  return pl.pallas_call(
      _tc2_body,
      grid=grid,
      in_specs=[
          pl.BlockSpec((_TC_R, D), lambda i: (i, 0)),
          pl.BlockSpec((NCORES, _TC_R, D), lambda i: (0, i, 0)),
          pl.BlockSpec((_TC_R, 1), lambda i: (i, 0)),
          _full((D, D)),
          _full((D, D)),
          _full((1, D)),
      ],
      out_specs=pl.BlockSpec((_TC_R, D), lambda i: (i, 0)),
      out_shape=jax.ShapeDtypeStruct((N, D), jnp.float32),
  )(h, parts, dinv, W_self, W_neigh, b.reshape(1, D))


def kernel(feat, edge_index, W_self0, W_neigh0, b0, W_self1, W_neigh1, b1,
           ln_g, ln_b):
  epw_real = E // NW
  pad = EPW - epw_real
  src = jnp.pad(edge_index[0].reshape(NW, epw_real), ((0, 0), (0, pad)),
                constant_values=0).reshape(NW, CH, CHUNK)
  dst = jnp.pad(edge_index[1].reshape(NW, epw_real), ((0, 0), (0, pad)),
                constant_values=N).reshape(NW, CH, CHUNK)

  feat_ext = jnp.concatenate(
      [feat, jnp.ones((N, 1), jnp.float32), jnp.zeros((N, WEXT - D - 1),
                                                      jnp.float32)], axis=1)
  z_ext = jnp.zeros((ROWS_PER_TILE, WEXT), jnp.float32)
  z_d = jnp.zeros((ROWS_PER_TILE, D), jnp.float32)

  parts1 = _sc_agg_ext(feat_ext, src, dst, z_ext)
  h1, dinv = _tc_layer1(feat, parts1, W_self0, W_neigh0, b0, ln_g, ln_b)
  parts2 = _sc_agg_d(h1, src, dst, z_d)
  return _tc_layer2(h1, parts2, dinv, W_self1, W_neigh1, b1)
